# baseline (speedup 1.0000x reference)
import jax
import jax.numpy as jnp
from jax import lax
from jax.experimental import pallas as pl
from jax.experimental.pallas import tpu as pltpu

VOCAB = 1000
TOKENS_PER_BLOCK = 1024


def _onehot_block(idx_ref, out_ref):
    idx = idx_ref[...]
    iota = lax.broadcasted_iota(jnp.int32, out_ref.shape, 1)
    out_ref[...] = (iota == idx).astype(jnp.float32)


def kernel(inputs):
    rows, cols = inputs.shape
    n = rows * cols
    t = TOKENS_PER_BLOCK
    grid = n // t
    idx_flat = inputs.reshape(n, 1)
    out = pl.pallas_call(
        _onehot_block,
        grid=(grid,),
        in_specs=[pl.BlockSpec((t, 1), lambda i: (i, 0))],
        out_specs=pl.BlockSpec((t, VOCAB), lambda i: (i, 0)),
        out_shape=jax.ShapeDtypeStruct((n, VOCAB), jnp.float32),
        compiler_params=pltpu.CompilerParams(
            dimension_semantics=("parallel",),
        ),
    )(idx_flat)
    return out.reshape(rows, cols, VOCAB)


# speedup vs baseline: 1.0064x; 1.0064x over previous
import jax
import jax.numpy as jnp
from jax import lax
from jax.experimental import pallas as pl
from jax.experimental.pallas import tpu as pltpu

VOCAB = 1000
T = 1024
K = 4


def _onehot_body(idx_ref, out_hbm, vmem, sems):
    i = pl.program_id(0)
    g = pl.num_programs(0)
    slot = lax.rem(i, K)

    @pl.when(i >= K)
    def _wait_prev():
        pltpu.make_async_copy(
            vmem.at[slot], out_hbm.at[pl.ds((i - K) * T, T), :], sems.at[slot]
        ).wait()

    idx = idx_ref[...]
    iota = lax.broadcasted_iota(jnp.int32, (T, VOCAB), 1)
    vmem[slot] = (iota == idx).astype(jnp.float32)

    pltpu.make_async_copy(
        vmem.at[slot], out_hbm.at[pl.ds(i * T, T), :], sems.at[slot]
    ).start()

    @pl.when(i == g - 1)
    def _drain():
        for j in range(K):
            step = i - (K - 1) + j

            @pl.when(step >= 0)
            def _():
                s = lax.rem(step, K)
                pltpu.make_async_copy(
                    vmem.at[s], out_hbm.at[pl.ds(step * T, T), :], sems.at[s]
                ).wait()


def kernel(inputs):
    rows, cols = inputs.shape
    n = rows * cols
    grid = n // T
    idx_flat = inputs.reshape(n, 1)
    out = pl.pallas_call(
        _onehot_body,
        grid=(grid,),
        in_specs=[pl.BlockSpec((T, 1), lambda i: (i, 0))],
        out_specs=pl.BlockSpec(memory_space=pl.ANY),
        out_shape=jax.ShapeDtypeStruct((n, VOCAB), jnp.float32),
        scratch_shapes=[
            pltpu.VMEM((K, T, VOCAB), jnp.float32),
            pltpu.SemaphoreType.DMA((K,)),
        ],
        compiler_params=pltpu.CompilerParams(
            dimension_semantics=("arbitrary",),
        ),
    )(idx_flat)
    return out.reshape(rows, cols, VOCAB)


# speedup vs baseline: 1.4592x; 1.4499x over previous
import jax
import jax.numpy as jnp
from jax import lax
from jax.experimental import pallas as pl
from jax.experimental.pallas import tpu as pltpu

VOCAB = 1000
B = 16


def _onehot_body(idx_ref, out_ref):
    idx = idx_ref[...]
    iota = lax.broadcasted_iota(jnp.int32, out_ref.shape, 2)
    out_ref[...] = (iota == idx[:, :, None]).astype(jnp.float32)


def kernel(inputs):
    rows, cols = inputs.shape
    return pl.pallas_call(
        _onehot_body,
        grid=(rows // B,),
        in_specs=[pl.BlockSpec((B, cols), lambda i: (i, 0))],
        out_specs=pl.BlockSpec((B, cols, VOCAB), lambda i: (i, 0, 0)),
        out_shape=jax.ShapeDtypeStruct((rows, cols, VOCAB), jnp.float32),
        compiler_params=pltpu.CompilerParams(
            dimension_semantics=("parallel",),
        ),
    )(inputs)
